# baseline (speedup 1.0000x reference)
import jax
import jax.numpy as jnp
from jax.experimental import pallas as pl
from jax.experimental.pallas import tpu as pltpu


def _pool_kernel(x_ref, m_ref, o_ref):
    m = m_ref[:, 0, :]
    cnt = jnp.sum((m != 0.0).astype(jnp.float32), axis=1, keepdims=True)
    x = x_ref[...].astype(jnp.float32)
    s = jnp.sum(x * m[:, :, None], axis=1)
    o_ref[...] = (s * (1.0 / cnt))[None].astype(o_ref.dtype)


def kernel(x, mask):
    assert x.ndim == 3, "expects (B, L, D) input pooled over dim=1"
    B, L, D = x.shape
    if mask.ndim == 3 and mask.shape[-1] == 1:
        mask2d = mask[:, :, 0]
    else:
        mask2d = mask
    assert mask2d.shape == (B, L)
    m_f32 = mask2d.astype(jnp.float32)[:, None, :]

    itemsize = jnp.dtype(x.dtype).itemsize
    slab = L * D * itemsize
    tb = max(1, (16 * 1024 * 1024) // slab)
    while tb > 1 and B % tb != 0:
        tb -= 1
    grid = (B // tb,)

    vmem_est = (
        2 * tb * slab
        + 2 * tb * L * 4
        + 3 * tb * L * D * 4
        + 2 * tb * D * itemsize
        + (2 << 20)
    )
    out = pl.pallas_call(
        _pool_kernel,
        out_shape=jax.ShapeDtypeStruct((B // tb, tb, D), x.dtype),
        grid=grid,
        in_specs=[
            pl.BlockSpec((tb, L, D), lambda b: (b, 0, 0)),
            pl.BlockSpec((tb, 1, L), lambda b: (b, 0, 0)),
        ],
        out_specs=pl.BlockSpec((1, tb, D), lambda b: (b, 0, 0)),
        compiler_params=pltpu.CompilerParams(
            dimension_semantics=("parallel",),
            vmem_limit_bytes=int(min(max(vmem_est, 16 << 20), 56 << 20)),
        ),
    )(x, m_f32)
    return out.reshape(B, D)
